# baseline (speedup 1.0000x reference)
import functools

import jax
import jax.numpy as jnp
from jax import lax
from jax.experimental import pallas as pl
from jax.experimental.pallas import tpu as pltpu
from jax.experimental.pallas import tpu_sc as plsc

N = 10000
E = 320000
D_IN = 128
D_HID = 128
D_OUT = 64

NC = 2
NS = 16
L = 16
NW = NC * NS
EPW = E // NW
C = 128
EPWP = 10240
NCH = EPWP // C
GB = 16
NBLK = NCH // GB
DUM = 10008
NA = N + 16
NP = 10240
ZC = 2048
RPW = 624
TAIL = N - NS * RPW

_BLK = 1000
_G = N // _BLK


def _make_edge_kernel(D, tc_tiling, NBUF):
    grp = D // L
    mesh = plsc.VectorSubcoreMesh(core_axis_name="c", subcore_axis_name="s")

    @functools.partial(
        pl.kernel,
        out_type=(
            jax.ShapeDtypeStruct((NC, N, D), jnp.float32),
            jax.ShapeDtypeStruct((NC, 1, NP), jnp.float32),
        ),
        mesh=mesh,
        compiler_params=pltpu.CompilerParams(
            needs_layout_passes=False,
            **({} if tc_tiling is None else
               {"use_tc_tiling_on_sc": tc_tiling})),
        scratch_types=[
            pltpu.VMEM((2, C), jnp.float32),
            pltpu.VMEM((2, C), jnp.float32),
            pltpu.VMEM((ZC,), jnp.float32),
            pltpu.VMEM((2, GB, C), jnp.int32),
            pltpu.VMEM((2, GB, C), jnp.int32),
            pltpu.VMEM((NBUF, C, D), jnp.float32),
            pltpu.VMEM((2, C), jnp.float32),
            pltpu.VMEM_SHARED((NA, D), jnp.float32),
            pltpu.VMEM_SHARED((NP,), jnp.float32),
            pltpu.SemaphoreType.DMA,
            pltpu.SemaphoreType.DMA,
            pltpu.SemaphoreType.DMA,
            pltpu.SemaphoreType.DMA,
            pltpu.SemaphoreType.DMA,
        ],
    )
    def edge_kernel(h_hbm, asrc_hbm, adst_hbm, eidx_hbm, part_hbm, dpart_hbm,
                    avs_v, avd_v, zden_v, sidx2_v, didx2_v, rows2_v, coef2_v,
                    acc_sh, den_sh, gsem, ssem, dsem, isem, asem):
        c = lax.axis_index("c")
        s = lax.axis_index("s")
        w = c * NS + s

        pltpu.async_copy(eidx_hbm.at[0, w, 0], sidx2_v.at[0], isem)
        pltpu.async_copy(eidx_hbm.at[1, w, 0], didx2_v.at[0], isem)

        zf = jnp.zeros((L,), jnp.float32)

        @pl.when(s == 0)
        def _zero_den():
            def zd(i, carry):
                zden_v[pl.ds(i * L, L)] = zf
                return carry

            lax.fori_loop(0, ZC // L, zd, 0)
            for j in range(NP // ZC):
                pltpu.sync_copy(zden_v, den_sh.at[pl.ds(j * ZC, ZC)])

        zrows = rows2_v.at[0]

        def zero_rows(i, carry):
            for r in range(D // L):
                zrows[i, pl.ds(r * L, L)] = zf
            return carry

        lax.fori_loop(0, C, zero_rows, 0)
        base = s * RPW
        for j in range(RPW // C):
            pltpu.sync_copy(zrows, acc_sh.at[pl.ds(base + j * C, C)])
        rem = RPW - (RPW // C) * C
        if rem:
            pltpu.sync_copy(zrows.at[pl.ds(0, rem)],
                            acc_sh.at[pl.ds(base + (RPW // C) * C, rem)])

        @pl.when(s == NS - 1)
        def _zero_tail():
            pltpu.sync_copy(zrows.at[pl.ds(0, TAIL)],
                            acc_sh.at[pl.ds(NS * RPW, TAIL)])

        plsc.subcore_barrier()

        def block(blk, carry):
            bb = lax.rem(blk, 2)
            sv = sidx2_v.at[bb]
            dv = didx2_v.at[bb]
            pltpu.make_async_copy(eidx_hbm.at[0, w, 0], sv, isem).wait()
            pltpu.make_async_copy(eidx_hbm.at[1, w, 0], dv, isem).wait()

            @pl.when(blk < NBLK - 1)
            def _stage_next():
                bn = jnp.minimum(blk + 1, NBLK - 1)
                pltpu.async_copy(eidx_hbm.at[0, w, bn], sidx2_v.at[1 - bb],
                                 isem)
                pltpu.async_copy(eidx_hbm.at[1, w, bn], didx2_v.at[1 - bb],
                                 isem)

            pltpu.async_copy(h_hbm.at[sv.at[0]], rows2_v.at[0], gsem)
            pltpu.async_copy(asrc_hbm.at[sv.at[0]], avs_v.at[0], asem)
            pltpu.async_copy(adst_hbm.at[dv.at[0]], avd_v.at[0], asem)

            def chunk(j, carry2):
                b = lax.rem(j, NBUF)
                b2 = lax.rem(j, 2)
                cb = coef2_v.at[b2]
                @pl.when(j >= 2)
                def _wait_old_den_scatter():
                    pltpu.make_async_copy(
                        cb, den_sh.at[dv.at[0]], dsem).wait()

                pltpu.make_async_copy(asrc_hbm.at[sv.at[0]], avs_v.at[b2],
                                      asem).wait()
                pltpu.make_async_copy(adst_hbm.at[dv.at[0]], avd_v.at[b2],
                                      asem).wait()
                for g in range(C // L):
                    a = (avs_v[b2, pl.ds(g * L, L)]
                         + avd_v[b2, pl.ds(g * L, L)])
                    a = jnp.where(a >= 0.0, a, a * 0.2)
                    e = jnp.exp(a)
                    cb[pl.ds(g * L, L)] = e

                pltpu.async_copy(cb, den_sh.at[dv.at[j]], dsem, add=True)

                @pl.when(j < GB - 1)
                def _start_next_alphas():
                    jn = jnp.minimum(j + 1, GB - 1)
                    nb2 = 1 - b2
                    pltpu.async_copy(asrc_hbm.at[sv.at[jn]], avs_v.at[nb2],
                                     asem)
                    pltpu.async_copy(adst_hbm.at[dv.at[jn]], avd_v.at[nb2],
                                     asem)

                tn = lax.rem(j + 1, NBUF)

                @pl.when(j >= NBUF - 1)
                def _wait_old_scatter():
                    pltpu.make_async_copy(
                        rows2_v.at[tn], acc_sh.at[dv.at[0]], ssem).wait()

                @pl.when(j < GB - 1)
                def _start_next_gather():
                    jn = jnp.minimum(j + 1, GB - 1)
                    pltpu.async_copy(h_hbm.at[sv.at[jn]], rows2_v.at[tn],
                                     gsem)

                pltpu.make_async_copy(
                    h_hbm.at[sv.at[0]], rows2_v.at[b], gsem).wait()
                rb = rows2_v.at[b]

                @plsc.parallel_loop(0, C, unroll=8)
                def _scale(ei):
                    ce = plsc.load_gather(
                        cb, [jnp.zeros((L,), jnp.int32) + ei])
                    for r in range(grp):
                        v = rb[ei, pl.ds(r * L, L)]
                        rb[ei, pl.ds(r * L, L)] = v * ce

                pltpu.async_copy(rows2_v.at[b], acc_sh.at[dv.at[j]], ssem,
                                 add=True)
                return carry2

            lax.fori_loop(0, GB, chunk, 0)
            for _ in range(NBUF - 1):
                pltpu.make_async_copy(
                    rows2_v.at[0], acc_sh.at[dv.at[0]], ssem).wait()
            for _ in range(2):
                pltpu.make_async_copy(
                    coef2_v.at[0], den_sh.at[dv.at[0]], dsem).wait()
            return carry

        lax.fori_loop(0, NBLK, block, 0)

        plsc.subcore_barrier()
        pltpu.sync_copy(acc_sh.at[pl.ds(base, RPW)],
                        part_hbm.at[c, pl.ds(base, RPW)])

        @pl.when(s == NS - 1)
        def _copy_tail():
            pltpu.sync_copy(acc_sh.at[pl.ds(NS * RPW, TAIL)],
                            part_hbm.at[c, pl.ds(NS * RPW, TAIL)])

        dstride = NP // NS
        pltpu.sync_copy(den_sh.at[pl.ds(s * dstride, dstride)],
                        dpart_hbm.at[c, 0, pl.ds(s * dstride, dstride)])

    return edge_kernel


_edge_kernel_l1 = _make_edge_kernel(D_HID, False, 2)
_edge_kernel_l2 = _make_edge_kernel(D_OUT, False, 3)


def _tc_first_body(x_ref, w_ref, a_ref, h_ref, al_ref):
    h = jnp.dot(x_ref[...], w_ref[...], preferred_element_type=jnp.float32)
    h_ref[...] = h
    al_ref[...] = jnp.dot(h, a_ref[...], preferred_element_type=jnp.float32)


def _tc_mid_body(p0_ref, p1_ref, dh_ref, b_ref, w_ref, a_ref, h_ref, al_ref):
    d = jnp.sum(dh_ref[...], axis=1)
    o = (p0_ref[...] + p1_ref[...]) / (d[:, None] + 1e-16) + b_ref[...]
    o = jnp.maximum(o, 0.0)
    h = jnp.dot(o, w_ref[...], preferred_element_type=jnp.float32)
    h_ref[...] = h
    al_ref[...] = jnp.dot(h, a_ref[...], preferred_element_type=jnp.float32)


def _tc_last_body(p0_ref, p1_ref, dh_ref, b_ref, out_ref):
    d = jnp.sum(dh_ref[...], axis=1)
    p = p0_ref[...] + p1_ref[...]
    o = p / (d[:, None] + 1e-16) + b_ref[...]
    m = jnp.max(o, axis=1, keepdims=True)
    ls = o - m
    out_ref[...] = ls - jnp.log(jnp.sum(jnp.exp(ls), axis=1, keepdims=True))


def _row_spec(d):
    return pl.BlockSpec((_BLK, d), lambda i: (i, 0))


def _full_spec(r, c):
    return pl.BlockSpec((r, c), lambda i: (0, 0))


_DH_SPEC = pl.BlockSpec((_BLK, NC), lambda i: (i, 0))


_tc_first = pl.pallas_call(
    _tc_first_body,
    grid=(_G,),
    in_specs=[_row_spec(D_IN), _full_spec(D_IN, D_HID), _full_spec(D_HID, 8)],
    out_specs=[_row_spec(D_HID), _row_spec(8)],
    out_shape=[
        jax.ShapeDtypeStruct((N, D_HID), jnp.float32),
        jax.ShapeDtypeStruct((N, 8), jnp.float32),
    ],
)

_tc_mid = pl.pallas_call(
    _tc_mid_body,
    grid=(_G,),
    in_specs=[_row_spec(D_HID), _row_spec(D_HID), _DH_SPEC,
              _full_spec(1, D_HID), _full_spec(D_HID, D_OUT),
              _full_spec(D_OUT, 8)],
    out_specs=[_row_spec(D_OUT), _row_spec(8)],
    out_shape=[
        jax.ShapeDtypeStruct((N, D_OUT), jnp.float32),
        jax.ShapeDtypeStruct((N, 8), jnp.float32),
    ],
)

_tc_last = pl.pallas_call(
    _tc_last_body,
    grid=(_G,),
    in_specs=[_row_spec(D_OUT), _row_spec(D_OUT), _DH_SPEC,
              _full_spec(1, D_OUT)],
    out_specs=_row_spec(D_OUT),
    out_shape=jax.ShapeDtypeStruct((N, D_OUT), jnp.float32),
)


def kernel(x, edge_index, W1, att_src1, att_dst1, b1, W2, att_src2, att_dst2, b2):
    A1 = jnp.concatenate(
        [att_src1[:, None], att_dst1[:, None],
         jnp.zeros((D_HID, 6), jnp.float32)], axis=1)
    A2 = jnp.concatenate(
        [att_src2[:, None], att_dst2[:, None],
         jnp.zeros((D_OUT, 6), jnp.float32)], axis=1)
    ei = edge_index.reshape(2, NW, EPW)
    fill = jnp.broadcast_to(jnp.array([[0], [DUM]], jnp.int32)[:, None, :],
                            (2, NW, EPWP - EPW))
    eidx = jnp.concatenate([ei, fill], axis=2).reshape(2, NW, NBLK, GB, C)

    h1, al1 = _tc_first(x, W1, A1)
    part1, dp1 = _edge_kernel_l1(
        h1, jnp.pad(al1[:, 0], (0, NA - N)), jnp.pad(al1[:, 1], (0, NA - N)),
        eidx)
    h2, al2 = _tc_mid(part1[0], part1[1], dp1[:, 0, :N].T, b1[None, :], W2, A2)
    part2, dp2 = _edge_kernel_l2(
        h2, jnp.pad(al2[:, 0], (0, NA - N)), jnp.pad(al2[:, 1], (0, NA - N)),
        eidx)
    return _tc_last(part2[0], part2[1], dp2[:, 0, :N].T, b2[None, :])


# speedup vs baseline: 2.0721x; 2.0721x over previous
import functools

import jax
import jax.numpy as jnp
from jax import lax
from jax.experimental import pallas as pl
from jax.experimental.pallas import tpu as pltpu
from jax.experimental.pallas import tpu_sc as plsc

N = 10000
E = 320000
D_IN = 128
D_HID = 128
D_OUT = 64

NC = 2
NS = 16
L = 16
NW = NC * NS
EPW = E // NW
C = 80
NCH = EPW // C
GB = 5
NBLK = NCH // GB
NP = 10240
ZC = 2048
RPW = 624
TAIL = N - NS * RPW

_BLK = 1000
_G = N // _BLK


def _make_edge_kernel(D, tc_tiling, NBUF):
    grp = D // L
    mesh = plsc.VectorSubcoreMesh(core_axis_name="c", subcore_axis_name="s")

    @functools.partial(
        pl.kernel,
        out_type=(
            jax.ShapeDtypeStruct((NC, N, D), jnp.float32),
            jax.ShapeDtypeStruct((NC, 1, NP), jnp.float32),
        ),
        mesh=mesh,
        compiler_params=pltpu.CompilerParams(
            needs_layout_passes=False,
            **({} if tc_tiling is None else
               {"use_tc_tiling_on_sc": tc_tiling})),
        scratch_types=[
            pltpu.VMEM((N,), jnp.float32),
            pltpu.VMEM((N,), jnp.float32),
            pltpu.VMEM((ZC,), jnp.float32),
            pltpu.VMEM((2, GB, C), jnp.int32),
            pltpu.VMEM((2, GB, C), jnp.int32),
            pltpu.VMEM((NBUF, C, D), jnp.float32),
            pltpu.VMEM((2, C), jnp.float32),
            pltpu.VMEM_SHARED((N, D), jnp.float32),
            pltpu.VMEM_SHARED((NP,), jnp.float32),
            pltpu.SemaphoreType.DMA,
            pltpu.SemaphoreType.DMA,
            pltpu.SemaphoreType.DMA,
            pltpu.SemaphoreType.DMA,
        ],
    )
    def edge_kernel(h_hbm, asrc_hbm, adst_hbm, eidx_hbm, part_hbm, dpart_hbm,
                    asrc_v, adst_v, zden_v, sidx2_v, didx2_v, rows2_v, coef2_v,
                    acc_sh, den_sh, gsem, ssem, dsem, isem):
        c = lax.axis_index("c")
        s = lax.axis_index("s")
        w = c * NS + s

        pltpu.async_copy(asrc_hbm, asrc_v, gsem)
        pltpu.async_copy(adst_hbm, adst_v, gsem)
        pltpu.async_copy(eidx_hbm.at[0, w, 0], sidx2_v.at[0], isem)
        pltpu.async_copy(eidx_hbm.at[1, w, 0], didx2_v.at[0], isem)

        zf = jnp.zeros((L,), jnp.float32)

        @pl.when(s == 0)
        def _zero_den():
            def zd(i, carry):
                zden_v[pl.ds(i * L, L)] = zf
                return carry

            lax.fori_loop(0, ZC // L, zd, 0)
            for j in range(NP // ZC):
                pltpu.sync_copy(zden_v, den_sh.at[pl.ds(j * ZC, ZC)])

        zrows = rows2_v.at[0]

        def zero_rows(i, carry):
            for r in range(D // L):
                zrows[i, pl.ds(r * L, L)] = zf
            return carry

        lax.fori_loop(0, C, zero_rows, 0)
        base = s * RPW
        for j in range(RPW // C):
            pltpu.sync_copy(zrows, acc_sh.at[pl.ds(base + j * C, C)])
        rem = RPW - (RPW // C) * C
        if rem:
            pltpu.sync_copy(zrows.at[pl.ds(0, rem)],
                            acc_sh.at[pl.ds(base + (RPW // C) * C, rem)])

        @pl.when(s == NS - 1)
        def _zero_tail():
            pltpu.sync_copy(zrows.at[pl.ds(0, TAIL)],
                            acc_sh.at[pl.ds(NS * RPW, TAIL)])

        pltpu.make_async_copy(asrc_hbm, asrc_v, gsem).wait()
        pltpu.make_async_copy(adst_hbm, adst_v, gsem).wait()

        plsc.subcore_barrier()

        def block(blk, carry):
            bb = lax.rem(blk, 2)
            sv = sidx2_v.at[bb]
            dv = didx2_v.at[bb]
            pltpu.make_async_copy(eidx_hbm.at[0, w, 0], sv, isem).wait()
            pltpu.make_async_copy(eidx_hbm.at[1, w, 0], dv, isem).wait()

            @pl.when(blk < NBLK - 1)
            def _stage_next():
                bn = jnp.minimum(blk + 1, NBLK - 1)
                pltpu.async_copy(eidx_hbm.at[0, w, bn], sidx2_v.at[1 - bb],
                                 isem)
                pltpu.async_copy(eidx_hbm.at[1, w, bn], didx2_v.at[1 - bb],
                                 isem)

            pltpu.async_copy(h_hbm.at[sv.at[0]], rows2_v.at[0], gsem)

            def chunk(j, carry2):
                b = lax.rem(j, NBUF)
                b2 = lax.rem(j, 2)
                cb = coef2_v.at[b2]
                @pl.when(j >= 2)
                def _wait_old_den_scatter():
                    pltpu.make_async_copy(
                        cb, den_sh.at[dv.at[0]], dsem).wait()

                for g in range(C // L):
                    si = sv[j, pl.ds(g * L, L)]
                    di = dv[j, pl.ds(g * L, L)]
                    a = (plsc.load_gather(asrc_v, [si])
                         + plsc.load_gather(adst_v, [di]))
                    a = jnp.where(a >= 0.0, a, a * 0.2)
                    e = jnp.exp(a)
                    cb[pl.ds(g * L, L)] = e

                pltpu.async_copy(cb, den_sh.at[dv.at[j]], dsem, add=True)

                tn = lax.rem(j + 1, NBUF)

                @pl.when(j >= NBUF - 1)
                def _wait_old_scatter():
                    pltpu.make_async_copy(
                        rows2_v.at[tn], acc_sh.at[dv.at[0]], ssem).wait()

                @pl.when(j < GB - 1)
                def _start_next_gather():
                    jn = jnp.minimum(j + 1, GB - 1)
                    pltpu.async_copy(h_hbm.at[sv.at[jn]], rows2_v.at[tn],
                                     gsem)

                pltpu.make_async_copy(
                    h_hbm.at[sv.at[0]], rows2_v.at[b], gsem).wait()
                rb = rows2_v.at[b]

                @plsc.parallel_loop(0, C, unroll=8)
                def _scale(ei):
                    ce = plsc.load_gather(
                        cb, [jnp.zeros((L,), jnp.int32) + ei])
                    for r in range(grp):
                        v = rb[ei, pl.ds(r * L, L)]
                        rb[ei, pl.ds(r * L, L)] = v * ce

                pltpu.async_copy(rows2_v.at[b], acc_sh.at[dv.at[j]], ssem,
                                 add=True)
                return carry2

            lax.fori_loop(0, GB, chunk, 0)
            for _ in range(NBUF - 1):
                pltpu.make_async_copy(
                    rows2_v.at[0], acc_sh.at[dv.at[0]], ssem).wait()
            for _ in range(2):
                pltpu.make_async_copy(
                    coef2_v.at[0], den_sh.at[dv.at[0]], dsem).wait()
            return carry

        lax.fori_loop(0, NBLK, block, 0)

        plsc.subcore_barrier()
        pltpu.sync_copy(acc_sh.at[pl.ds(base, RPW)],
                        part_hbm.at[c, pl.ds(base, RPW)])

        @pl.when(s == NS - 1)
        def _copy_tail():
            pltpu.sync_copy(acc_sh.at[pl.ds(NS * RPW, TAIL)],
                            part_hbm.at[c, pl.ds(NS * RPW, TAIL)])

        dstride = NP // NS
        pltpu.sync_copy(den_sh.at[pl.ds(s * dstride, dstride)],
                        dpart_hbm.at[c, 0, pl.ds(s * dstride, dstride)])

    return edge_kernel


_edge_kernel_l1 = _make_edge_kernel(D_HID, False, 2)
_edge_kernel_l2 = _make_edge_kernel(D_OUT, False, 3)


def _tc_first_body(x_ref, w_ref, a_ref, h_ref, al_ref):
    h = jnp.dot(x_ref[...], w_ref[...], preferred_element_type=jnp.float32)
    h_ref[...] = h
    al_ref[...] = jnp.dot(h, a_ref[...], preferred_element_type=jnp.float32)


def _tc_mid_body(p0_ref, p1_ref, dh_ref, b_ref, w_ref, a_ref, h_ref, al_ref):
    d = jnp.sum(dh_ref[...], axis=1)
    o = (p0_ref[...] + p1_ref[...]) / (d[:, None] + 1e-16) + b_ref[...]
    o = jnp.maximum(o, 0.0)
    h = jnp.dot(o, w_ref[...], preferred_element_type=jnp.float32)
    h_ref[...] = h
    al_ref[...] = jnp.dot(h, a_ref[...], preferred_element_type=jnp.float32)


def _tc_last_body(p0_ref, p1_ref, dh_ref, b_ref, out_ref):
    d = jnp.sum(dh_ref[...], axis=1)
    p = p0_ref[...] + p1_ref[...]
    o = p / (d[:, None] + 1e-16) + b_ref[...]
    m = jnp.max(o, axis=1, keepdims=True)
    ls = o - m
    out_ref[...] = ls - jnp.log(jnp.sum(jnp.exp(ls), axis=1, keepdims=True))


def _row_spec(d):
    return pl.BlockSpec((_BLK, d), lambda i: (i, 0))


def _full_spec(r, c):
    return pl.BlockSpec((r, c), lambda i: (0, 0))


_DH_SPEC = pl.BlockSpec((_BLK, NC), lambda i: (i, 0))


_tc_first = pl.pallas_call(
    _tc_first_body,
    grid=(_G,),
    in_specs=[_row_spec(D_IN), _full_spec(D_IN, D_HID), _full_spec(D_HID, 8)],
    out_specs=[_row_spec(D_HID), _row_spec(8)],
    out_shape=[
        jax.ShapeDtypeStruct((N, D_HID), jnp.float32),
        jax.ShapeDtypeStruct((N, 8), jnp.float32),
    ],
)

_tc_mid = pl.pallas_call(
    _tc_mid_body,
    grid=(_G,),
    in_specs=[_row_spec(D_HID), _row_spec(D_HID), _DH_SPEC,
              _full_spec(1, D_HID), _full_spec(D_HID, D_OUT),
              _full_spec(D_OUT, 8)],
    out_specs=[_row_spec(D_OUT), _row_spec(8)],
    out_shape=[
        jax.ShapeDtypeStruct((N, D_OUT), jnp.float32),
        jax.ShapeDtypeStruct((N, 8), jnp.float32),
    ],
)

_tc_last = pl.pallas_call(
    _tc_last_body,
    grid=(_G,),
    in_specs=[_row_spec(D_OUT), _row_spec(D_OUT), _DH_SPEC,
              _full_spec(1, D_OUT)],
    out_specs=_row_spec(D_OUT),
    out_shape=jax.ShapeDtypeStruct((N, D_OUT), jnp.float32),
)


def kernel(x, edge_index, W1, att_src1, att_dst1, b1, W2, att_src2, att_dst2, b2):
    A1 = jnp.concatenate(
        [att_src1[:, None], att_dst1[:, None],
         jnp.zeros((D_HID, 6), jnp.float32)], axis=1)
    A2 = jnp.concatenate(
        [att_src2[:, None], att_dst2[:, None],
         jnp.zeros((D_OUT, 6), jnp.float32)], axis=1)
    eidx = edge_index.reshape(2, NW, NBLK, GB, C)

    h1, al1 = _tc_first(x, W1, A1)
    part1, dp1 = _edge_kernel_l1(h1, al1[:, 0], al1[:, 1], eidx)
    h2, al2 = _tc_mid(part1[0], part1[1], dp1[:, 0, :N].T, b1[None, :], W2, A2)
    part2, dp2 = _edge_kernel_l2(h2, al2[:, 0], al2[:, 1], eidx)
    return _tc_last(part2[0], part2[1], dp2[:, 0, :N].T, b2[None, :])
